# Initial kernel scaffold; baseline (speedup 1.0000x reference)
import jax
import jax.numpy as jnp
from jax.experimental import pallas as pl


def kernel(fg_indices, fg_embedding):
    raise NotImplementedError("write your pallas kernel here")


# speedup vs baseline: 8.0557x; 8.0557x over previous
import functools

import jax
import jax.numpy as jnp
from jax import lax
from jax.experimental import pallas as pl
from jax.experimental.pallas import tpu as pltpu
from jax.experimental.pallas import tpu_sc as plsc

B = 16384
LF = 20
V = 1000
D = 133
DP = 144
NV = DP // 16
NC, NS = 2, 16
NW = NC * NS
BW = B // NW
C = 32
GPC = C * LF // 128


@functools.partial(
    pl.kernel,
    mesh=plsc.VectorSubcoreMesh(core_axis_name="c", subcore_axis_name="s"),
    out_type=jax.ShapeDtypeStruct((B * DP,), jnp.float32),
    compiler_params=pltpu.CompilerParams(use_tc_tiling_on_sc=False),
    scratch_types=[
        pltpu.VMEM((GPC * 128,), jnp.int32),
        pltpu.VMEM((C * LF, DP), jnp.float32),
        pltpu.VMEM((C * DP,), jnp.float32),
        pltpu.SemaphoreType.DMA,
    ],
)
def _fg_pool(idx_hbm, tab_hbm, out_hbm, idx_v, rows_v, obuf, sem):
    wid = lax.axis_index("s") * NC + lax.axis_index("c")
    inv = jnp.float32(1.0 / LF)

    def chunk(i, _):
        base = wid * BW + i * C
        pltpu.sync_copy(idx_hbm.at[pl.ds(base * LF, C * LF)], idx_v)
        descs = [
            pltpu.async_copy(tab_hbm.at[idx_v.at[pl.ds(j * 128, 128)]],
                             rows_v.at[pl.ds(j * 128, 128)], sem)
            for j in range(GPC)
        ]
        for d in descs:
            d.wait()

        def row_body(r, _):
            def g_body(g, accs):
                row = r * LF + g
                return tuple(a + rows_v[row, pl.ds(16 * v, 16)]
                             for v, a in enumerate(accs))
            accs = lax.fori_loop(
                0, LF, g_body,
                tuple(jnp.zeros((16,), jnp.float32) for _ in range(NV)))
            for v in range(NV):
                obuf[pl.ds(r * DP + 16 * v, 16)] = accs[v] * inv
            return 0

        lax.fori_loop(0, C, row_body, 0)
        pltpu.sync_copy(obuf, out_hbm.at[pl.ds(base * DP, C * DP)])
        return 0

    lax.fori_loop(0, BW // C, chunk, 0)


def kernel(fg_indices, fg_embedding):
    idx = fg_indices.astype(jnp.int32).reshape(B * LF)
    tab = jnp.pad(fg_embedding, ((0, 0), (0, DP - D)))
    out = _fg_pool(idx, tab)
    return out.reshape(B, DP)[:, :D]


# speedup vs baseline: 13.7876x; 1.7115x over previous
import functools

import jax
import jax.numpy as jnp
from jax import lax
from jax.experimental import pallas as pl
from jax.experimental.pallas import tpu as pltpu
from jax.experimental.pallas import tpu_sc as plsc

B = 16384
LF = 20
V = 1000
D = 133
DP = 144
W = DP // 2
NC, NS = 2, 16
NW = NC * NS
BW = B // NW
C = 64
OFFS = (0, 16, 32, 48, 56)


@functools.partial(
    pl.kernel,
    mesh=plsc.VectorSubcoreMesh(core_axis_name="c", subcore_axis_name="s"),
    out_type=jax.ShapeDtypeStruct((B * W,), jnp.int32),
    compiler_params=pltpu.CompilerParams(use_tc_tiling_on_sc=False,
                                         needs_layout_passes=False),
    scratch_types=[
        pltpu.VMEM((V * W,), jnp.int32),
        pltpu.VMEM((C * 32,), jnp.int32),
        pltpu.VMEM((C * W,), jnp.int32),
    ],
)
def _fg_pool(idx_hbm, tab_hbm, out_hbm, tab_v, idx_v, obuf):
    wid = lax.axis_index("s") * NC + lax.axis_index("c")
    pltpu.sync_copy(tab_hbm, tab_v)

    def chunk(i, _):
        base = wid * BW + i * C
        pltpu.sync_copy(idx_hbm.at[pl.ds(base * 32, C * 32)], idx_v)

        def row_body(r, _):
            iv0 = idx_v[pl.ds(r * 32, 16)]
            iv1 = idx_v[pl.ds(r * 32 + 16, 16)]
            accs = [jnp.zeros((32,), jnp.bfloat16) for _ in OFFS]
            for g in range(LF):
                rowoff = (iv0[g] if g < 16 else iv1[g - 16]) * W
                for v, off in enumerate(OFFS):
                    accs[v] += plsc.bitcast(
                        tab_v[pl.ds(rowoff + off, 16)], jnp.bfloat16)
            for v, off in enumerate(OFFS):
                obuf[pl.ds(r * W + off, 16)] = plsc.bitcast(accs[v], jnp.int32)
            return 0

        lax.fori_loop(0, C, row_body, 0)
        pltpu.sync_copy(obuf, out_hbm.at[pl.ds(base * W, C * W)])
        return 0

    lax.fori_loop(0, BW // C, chunk, 0)


def kernel(fg_indices, fg_embedding):
    idx = jnp.pad(fg_indices.astype(jnp.int32),
                  ((0, 0), (0, 32 - LF))).reshape(B * 32)
    tab = jnp.pad(fg_embedding, ((0, 0), (0, DP - D))).astype(jnp.bfloat16)
    packed = lax.bitcast_convert_type(tab.reshape(V, W, 2), jnp.int32)
    out = _fg_pool(idx, packed.reshape(V * W))
    sums = lax.bitcast_convert_type(
        out.reshape(B, W), jnp.bfloat16).reshape(B, DP)
    return (sums[:, :D].astype(jnp.float32)) * jnp.float32(1.0 / LF)


# speedup vs baseline: 15.6561x; 1.1355x over previous
import functools

import jax
import jax.numpy as jnp
from jax import lax
from jax.experimental import pallas as pl
from jax.experimental.pallas import tpu as pltpu
from jax.experimental.pallas import tpu_sc as plsc

B = 16384
LF = 20
V = 1000
D = 133
DP = 144
W = 80
NC, NS = 2, 16
NW = NC * NS
BW = B // NW
C = 64
NVP = W // 16


@functools.partial(
    pl.kernel,
    mesh=plsc.VectorSubcoreMesh(core_axis_name="c", subcore_axis_name="s"),
    out_type=jax.ShapeDtypeStruct((B * DP,), jnp.float32),
    compiler_params=pltpu.CompilerParams(use_tc_tiling_on_sc=False,
                                         needs_layout_passes=False),
    scratch_types=[
        pltpu.VMEM((V * W,), jnp.int32),
        pltpu.VMEM((C * LF,), jnp.int32),
        pltpu.VMEM((C * DP,), jnp.float32),
    ],
)
def _fg_pool(idx_hbm, tab_hbm, out_hbm, tab_v, idx_v, obuf):
    wid = lax.axis_index("s") * NC + lax.axis_index("c")
    inv = jnp.float32(1.0 / LF)
    pltpu.sync_copy(tab_hbm, tab_v)

    def chunk(i, _):
        base = wid * BW + i * C
        pltpu.sync_copy(idx_hbm.at[pl.ds(base * LF, C * LF)], idx_v)

        def group_body(q, _):
            ivs = [idx_v[pl.ds(q * 80 + 16 * k, 16)] for k in range(5)]
            for j in range(4):
                accs = [jnp.zeros((32,), jnp.bfloat16) for _ in range(NVP)]
                for g in range(LF):
                    p = 20 * j + g
                    rowoff = ivs[p // 16][p % 16] * W
                    for v in range(NVP):
                        accs[v] += plsc.bitcast(
                            tab_v[pl.ds(rowoff + 16 * v, 16)], jnp.bfloat16)
                obase = (q * 4 + j) * DP
                for v in range(NVP):
                    lo, hi = plsc.unpack(accs[v],
                                         format=plsc.PackFormat.INTERLEAVED)
                    obuf[pl.ds(obase + 32 * v, 16)] = lo * inv
                    if v < NVP - 1:
                        obuf[pl.ds(obase + 32 * v + 16, 16)] = hi * inv
            return 0

        lax.fori_loop(0, C // 4, group_body, 0)
        pltpu.sync_copy(obuf, out_hbm.at[pl.ds(base * DP, C * DP)])
        return 0

    lax.fori_loop(0, BW // C, chunk, 0)


def kernel(fg_indices, fg_embedding):
    idx = fg_indices.astype(jnp.int32).reshape(B * LF)
    tab = jnp.pad(fg_embedding, ((0, 0), (0, 160 - D))).astype(jnp.bfloat16)
    folded = tab.reshape(V, NVP, 2, 16).transpose(0, 1, 3, 2)
    packed = lax.bitcast_convert_type(folded, jnp.int32)
    out = _fg_pool(idx, packed.reshape(V * W))
    return out.reshape(B, DP)[:, :D]


# speedup vs baseline: 17.3986x; 1.1113x over previous
import functools

import jax
import jax.numpy as jnp
from jax import lax
from jax.experimental import pallas as pl
from jax.experimental.pallas import tpu as pltpu
from jax.experimental.pallas import tpu_sc as plsc

B = 16384
LF = 20
V = 1000
D = 133
DP = 144
W = 80
NC, NS = 2, 16
NW = NC * NS
BW = B // NW
C = 64
NVP = W // 16


@functools.partial(
    pl.kernel,
    mesh=plsc.VectorSubcoreMesh(core_axis_name="c", subcore_axis_name="s"),
    out_type=jax.ShapeDtypeStruct((B, D), jnp.float32),
    compiler_params=pltpu.CompilerParams(use_tc_tiling_on_sc=True,
                                         needs_layout_passes=False),
    scratch_types=[
        pltpu.VMEM((V * W,), jnp.int32),
        pltpu.VMEM((C * LF,), jnp.int32),
        pltpu.VMEM((C, D), jnp.float32),
    ],
)
def _fg_pool(idx_hbm, tab_hbm, out_hbm, tab_v, idx_v, obuf):
    wid = lax.axis_index("s") * NC + lax.axis_index("c")
    inv = jnp.float32(1.0 / LF)
    lane = lax.iota(jnp.int32, 16)
    tail_cols = jnp.minimum(lane + (D - 5), D - 1)
    tail_mask = lane < 5
    pltpu.sync_copy(tab_hbm, tab_v)

    def chunk(i, _):
        base = wid * BW + i * C
        pltpu.sync_copy(idx_hbm.at[pl.ds(base * LF, C * LF)], idx_v)

        def group_body(q, _):
            ivs = [idx_v[pl.ds(q * 80 + 16 * k, 16)] for k in range(5)]
            for j in range(4):
                accs = [jnp.zeros((32,), jnp.bfloat16) for _ in range(NVP)]
                for g in range(LF):
                    p = 20 * j + g
                    rowoff = ivs[p // 16][p % 16] * W
                    for v in range(NVP):
                        accs[v] += plsc.bitcast(
                            tab_v[pl.ds(rowoff + 16 * v, 16)], jnp.bfloat16)
                r = q * 4 + j
                for v in range(NVP):
                    lo, hi = plsc.unpack(accs[v],
                                         format=plsc.PackFormat.INTERLEAVED)
                    if v < NVP - 1:
                        obuf[r, pl.ds(32 * v, 16)] = lo * inv
                        obuf[r, pl.ds(32 * v + 16, 16)] = hi * inv
                    else:
                        plsc.store_scatter(
                            obuf, [jnp.full((16,), r, jnp.int32), tail_cols],
                            lo * inv, mask=tail_mask)
            return 0

        lax.fori_loop(0, C // 4, group_body, 0)
        pltpu.sync_copy(obuf, out_hbm.at[pl.ds(base, C), :])
        return 0

    lax.fori_loop(0, BW // C, chunk, 0)


def kernel(fg_indices, fg_embedding):
    idx = fg_indices.astype(jnp.int32).reshape(B * LF)
    tab = jnp.pad(fg_embedding, ((0, 0), (0, 160 - D))).astype(jnp.bfloat16)
    folded = tab.reshape(V, NVP, 2, 16).transpose(0, 1, 3, 2)
    packed = lax.bitcast_convert_type(folded, jnp.int32)
    return _fg_pool(idx, packed.reshape(V * W))


# speedup vs baseline: 17.9498x; 1.0317x over previous
import functools

import jax
import jax.numpy as jnp
from jax import lax
from jax.experimental import pallas as pl
from jax.experimental.pallas import tpu as pltpu
from jax.experimental.pallas import tpu_sc as plsc

B = 16384
LF = 20
V = 1000
D = 133
NP = 67
NC, NS = 2, 16
NW = NC * NS
BW = B // NW
CB = 128


@functools.partial(
    pl.kernel,
    mesh=plsc.VectorSubcoreMesh(core_axis_name="c", subcore_axis_name="s"),
    out_type=jax.ShapeDtypeStruct((D, B), jnp.float32),
    compiler_params=pltpu.CompilerParams(use_tc_tiling_on_sc=True,
                                         needs_layout_passes=False),
    scratch_types=[
        pltpu.VMEM((NP * V,), jnp.int32),
        pltpu.VMEM((LF, CB), jnp.int32),
        pltpu.VMEM((D, CB), jnp.float32),
    ],
)
def _fg_pool(idx_hbm, tab_hbm, out_hbm, tab_v, idx_v, obuf):
    wid = lax.axis_index("s") * NC + lax.axis_index("c")
    inv = jnp.float32(1.0 / LF)
    pltpu.sync_copy(tab_hbm, tab_v)

    def chunk(i, _):
        b0 = wid * BW + i * CB
        pltpu.sync_copy(idx_hbm.at[:, pl.ds(b0, CB)], idx_v)

        def group(qb, _):
            bb = qb * 16
            idxs = [idx_v[g, pl.ds(bb, 16)] for g in range(LF)]

            def accum(cp):
                word = tab_v.at[pl.ds(cp * V, V)]
                acc = jnp.zeros((32,), jnp.bfloat16)
                for g in range(LF):
                    acc += plsc.bitcast(
                        plsc.load_gather(word, [idxs[g]]), jnp.bfloat16)
                return plsc.unpack(acc, format=plsc.PackFormat.INTERLEAVED)

            def pair_body(cp, _):
                lo, hi = accum(cp)
                obuf[2 * cp, pl.ds(bb, 16)] = lo * inv
                obuf[2 * cp + 1, pl.ds(bb, 16)] = hi * inv
                return 0

            lax.fori_loop(0, NP - 1, pair_body, 0)
            lo, _ = accum(NP - 1)
            obuf[D - 1, pl.ds(bb, 16)] = lo * inv
            return 0

        lax.fori_loop(0, CB // 16, group, 0)
        pltpu.sync_copy(obuf, out_hbm.at[:, pl.ds(b0, CB)])
        return 0

    lax.fori_loop(0, BW // CB, chunk, 0)


def kernel(fg_indices, fg_embedding):
    idx_t = fg_indices.astype(jnp.int32).T
    tab_t = jnp.pad(fg_embedding.T, ((0, 1), (0, 0)))
    folded = tab_t.astype(jnp.bfloat16).reshape(NP, 2, V).transpose(0, 2, 1)
    packed = lax.bitcast_convert_type(folded, jnp.int32)
    out_t = _fg_pool(idx_t, packed.reshape(NP * V))
    return out_t.T


# speedup vs baseline: 21.2707x; 1.1850x over previous
import functools

import jax
import jax.numpy as jnp
from jax import lax
from jax.experimental import pallas as pl
from jax.experimental.pallas import tpu as pltpu
from jax.experimental.pallas import tpu_sc as plsc

B = 16384
LF = 20
V = 1000
D = 133
NP = 67
NC, NS = 2, 16
NW = NC * NS
BW = B // NW
CB = 128


@functools.partial(
    pl.kernel,
    mesh=plsc.VectorSubcoreMesh(core_axis_name="c", subcore_axis_name="s"),
    out_type=jax.ShapeDtypeStruct((D, B), jnp.float32),
    compiler_params=pltpu.CompilerParams(use_tc_tiling_on_sc=True,
                                         needs_layout_passes=False),
    scratch_types=[
        pltpu.VMEM((NP * V,), jnp.int32),
        pltpu.VMEM((LF, CB), jnp.int32),
        pltpu.VMEM((D, CB), jnp.float32),
    ],
)
def _fg_pool(idx_hbm, tab_hbm, out_hbm, tab_v, idx_v, obuf):
    wid = lax.axis_index("s") * NC + lax.axis_index("c")
    inv = jnp.float32(1.0 / LF)
    pltpu.sync_copy(tab_hbm, tab_v)

    def chunk(i, _):
        b0 = wid * BW + i * CB
        pltpu.sync_copy(idx_hbm.at[:, pl.ds(b0, CB)], idx_v)

        def group(qb, _):
            bb = qb * 16
            idxs = [idx_v[g, pl.ds(bb, 16)] for g in range(LF)]

            def accum(cp):
                word = tab_v.at[pl.ds(cp * V, V)]
                vals = [plsc.bitcast(plsc.load_gather(word, [idxs[g]]),
                                     jnp.bfloat16) for g in range(LF)]
                while len(vals) > 1:
                    vals = [vals[k] + vals[k + 1]
                            for k in range(0, len(vals) - 1, 2)] + (
                        [vals[-1]] if len(vals) % 2 else [])
                return plsc.unpack(vals[0],
                                   format=plsc.PackFormat.INTERLEAVED)

            def pair_body(u, _):
                for cp in (2 * u, 2 * u + 1):
                    lo, hi = accum(cp)
                    obuf[2 * cp, pl.ds(bb, 16)] = lo * inv
                    obuf[2 * cp + 1, pl.ds(bb, 16)] = hi * inv
                return 0

            lax.fori_loop(0, (NP - 1) // 2, pair_body, 0)
            lo, _ = accum(NP - 1)
            obuf[D - 1, pl.ds(bb, 16)] = lo * inv
            return 0

        lax.fori_loop(0, CB // 16, group, 0)
        pltpu.sync_copy(obuf, out_hbm.at[:, pl.ds(b0, CB)])
        return 0

    lax.fori_loop(0, BW // CB, chunk, 0)


def kernel(fg_indices, fg_embedding):
    idx_t = fg_indices.astype(jnp.int32).T
    tab_t = jnp.pad(fg_embedding.T, ((0, 1), (0, 0)))
    folded = tab_t.astype(jnp.bfloat16).reshape(NP, 2, V).transpose(0, 2, 1)
    packed = lax.bitcast_convert_type(folded, jnp.int32)
    out_t = _fg_pool(idx_t, packed.reshape(NP * V))
    return out_t.T


# speedup vs baseline: 21.7075x; 1.0205x over previous
import functools

import jax
import jax.numpy as jnp
from jax import lax
from jax.experimental import pallas as pl
from jax.experimental.pallas import tpu as pltpu
from jax.experimental.pallas import tpu_sc as plsc

B = 16384
LF = 20
V = 1000
D = 133
NP = 67
NC, NS = 2, 16
NW = NC * NS
BW = B // NW
CB = 128


@functools.partial(
    pl.kernel,
    mesh=plsc.VectorSubcoreMesh(core_axis_name="c", subcore_axis_name="s"),
    out_type=jax.ShapeDtypeStruct((D, B), jnp.float32),
    compiler_params=pltpu.CompilerParams(use_tc_tiling_on_sc=True,
                                         needs_layout_passes=False),
    scratch_types=[
        pltpu.VMEM((NP * V,), jnp.int32),
        pltpu.VMEM((LF, CB), jnp.int32),
        pltpu.VMEM((D, CB), jnp.float32),
    ],
)
def _fg_pool(idx_hbm, tab_hbm, out_hbm, tab_v, idx_v, obuf):
    wid = lax.axis_index("s") * NC + lax.axis_index("c")
    pltpu.sync_copy(tab_hbm, tab_v)

    def chunk(i, _):
        b0 = wid * BW + i * CB
        pltpu.sync_copy(idx_hbm.at[:, pl.ds(b0, CB)], idx_v)

        def group(qb, _):
            bb = qb * 16
            idxs = [idx_v[g, pl.ds(bb, 16)] for g in range(LF)]

            def accum(cp):
                word = tab_v.at[pl.ds(cp * V, V)]
                vals = [plsc.bitcast(plsc.load_gather(word, [idxs[g]]),
                                     jnp.bfloat16) for g in range(LF)]
                while len(vals) > 1:
                    vals = [vals[k] + vals[k + 1]
                            for k in range(0, len(vals) - 1, 2)] + (
                        [vals[-1]] if len(vals) % 2 else [])
                return plsc.unpack(vals[0],
                                   format=plsc.PackFormat.INTERLEAVED)

            def pair_body(u, _):
                for cp in (3 * u, 3 * u + 1, 3 * u + 2):
                    lo, hi = accum(cp)
                    obuf[2 * cp, pl.ds(bb, 16)] = lo
                    obuf[2 * cp + 1, pl.ds(bb, 16)] = hi
                return 0

            lax.fori_loop(0, (NP - 1) // 3, pair_body, 0)
            lo, _ = accum(NP - 1)
            obuf[D - 1, pl.ds(bb, 16)] = lo
            return 0

        lax.fori_loop(0, CB // 16, group, 0)
        pltpu.sync_copy(obuf, out_hbm.at[:, pl.ds(b0, CB)])
        return 0

    lax.fori_loop(0, BW // CB, chunk, 0)


def kernel(fg_indices, fg_embedding):
    idx_t = fg_indices.astype(jnp.int32).T
    tab_t = jnp.pad(fg_embedding.T * jnp.float32(1.0 / LF),
                    ((0, 1), (0, 0)))
    folded = tab_t.astype(jnp.bfloat16).reshape(NP, 2, V).transpose(0, 2, 1)
    packed = lax.bitcast_convert_type(folded, jnp.int32)
    out_t = _fg_pool(idx_t, packed.reshape(NP * V))
    return out_t.T
